# baseline (speedup 1.0000x reference)
import functools

import jax
import jax.numpy as jnp
from jax import lax
from jax.experimental import pallas as pl
from jax.experimental.pallas import tpu as pltpu
from jax.experimental.pallas import tpu_sc as plsc

N = 10000
E = 320000
D = 128

NC = 2
NS = 16
NW = NC * NS
NP = 10240
CB = 128
EPW = NP
NCHUNK = EPW // CB
RPT = NP // NS
HG = CB // 16


def _sc_aggregate(x, ei4, zeros):
  mesh = plsc.VectorSubcoreMesh(core_axis_name="c", subcore_axis_name="s")

  @functools.partial(
      pl.kernel,
      out_type=(jax.ShapeDtypeStruct((NC, NP, D), jnp.float32),
                jax.ShapeDtypeStruct((NW, NP), jnp.float32)),
      mesh=mesh,
      compiler_params=pltpu.CompilerParams(use_tc_tiling_on_sc=False,
                                           needs_layout_passes=False),
      scratch_types=[
          pltpu.VMEM((4, 2, CB), jnp.int32),
          pltpu.VMEM((CB, D), jnp.float32),
          pltpu.VMEM((CB, D), jnp.float32),
          pltpu.VMEM((NP,), jnp.float32),
          pltpu.VMEM_SHARED((NP, D), jnp.float32),
          pltpu.SemaphoreType.DMA,
          pltpu.SemaphoreType.DMA,
          [pltpu.SemaphoreType.DMA] * 4,
      ],
  )
  def agg_kernel(x_hbm, ei_hbm, z_hbm, acc_hbm, deg_hbm,
                 idxb, gbufa, gbufb, deg_v, acc_sh, sema, semb, semi):
    cid = lax.axis_index("c")
    sid = lax.axis_index("s")
    wid = cid * NS + sid

    pltpu.sync_copy(z_hbm.at[pl.ds(sid * RPT, RPT)],
                    acc_sh.at[pl.ds(sid * RPT, RPT)])

    z16 = jnp.zeros((16,), jnp.float32)

    @pl.loop(0, NP // 16)
    def _(i):
      deg_v[pl.ds(i * 16, 16)] = z16

    plsc.subcore_barrier()

    ones16 = jnp.ones((16,), jnp.float32)

    def hist(slot):
      for k in range(HG):
        idx = idxb[slot, 1, pl.ds(k * 16, 16)]
        plsc.addupdate_scatter(deg_v, [idx], ones16)

    def idx_wait(slot):
      pltpu.make_async_copy(ei_hbm.at[wid, 0], idxb.at[slot], semi[slot]).wait()

    def idx_prefetch(slot, c, J):
      @pl.when(J + 4 + slot < NCHUNK)
      def _():
        pltpu.async_copy(ei_hbm.at[wid, c + 4], idxb.at[slot], semi[slot])

    pltpu.async_copy(ei_hbm.at[wid, 0], idxb.at[0], semi[0])
    pltpu.async_copy(ei_hbm.at[wid, 1], idxb.at[1], semi[1])
    pltpu.async_copy(ei_hbm.at[wid, 2], idxb.at[2], semi[2])
    pltpu.async_copy(ei_hbm.at[wid, 3], idxb.at[3], semi[3])
    idx_wait(0)
    pltpu.async_copy(x_hbm.at[idxb.at[0, 0]], gbufa, sema)

    @pl.loop(0, NCHUNK, step=4)
    def _(J):
      for b in range(2):
        s0, s1 = 2 * b, 2 * b + 1
        c0 = J + 2 * b

        idx_wait(s1)
        hb = pltpu.async_copy(x_hbm.at[idxb.at[s1, 0]], gbufb, semb)
        pltpu.make_async_copy(x_hbm.at[pl.ds(0, CB)], gbufa, sema).wait()
        pltpu.sync_copy(gbufa, acc_sh.at[idxb.at[s0, 1]], add=True)
        hist(s0)
        idx_prefetch(s0, c0, J)

        nxt = (s1 + 1) % 4

        @pl.when(c0 + 2 < NCHUNK)
        def _():
          idx_wait(nxt)
          pltpu.async_copy(x_hbm.at[idxb.at[nxt, 0]], gbufa, sema)

        hb.wait()
        pltpu.sync_copy(gbufb, acc_sh.at[idxb.at[s1, 1]], add=True)
        hist(s1)
        idx_prefetch(s1, c0 + 1, J)

    plsc.subcore_barrier()

    pltpu.sync_copy(acc_sh.at[pl.ds(sid * RPT, RPT)],
                    acc_hbm.at[cid, pl.ds(sid * RPT, RPT)])
    pltpu.sync_copy(deg_v, deg_hbm.at[wid])

  return agg_kernel(x, ei4, zeros)


def _tc_finish_body(agg_ref, deg_ref, x_ref, wl_ref, bl_ref, wr_ref,
                    out_ref):
  a = agg_ref[0] + agg_ref[1]
  deg = jnp.sum(deg_ref[...], axis=1)[:, None]
  mean = a / jnp.maximum(deg, 1.0)
  out = (
      lax.dot_general(mean, wl_ref[...], (((1,), (1,)), ((), ())),
                      preferred_element_type=jnp.float32)
      + lax.dot_general(x_ref[...], wr_ref[...], (((1,), (1,)), ((), ())),
                        preferred_element_type=jnp.float32)
      + bl_ref[...]
  )
  norm = jnp.sqrt(jnp.sum(out * out, axis=-1, keepdims=True))
  out_ref[...] = out / jnp.maximum(norm, 1e-12)


def _tc_finish(agg2, deg2, x, W_l, b_l2, W_r):
  blk = 2000
  grid = N // blk
  return pl.pallas_call(
      _tc_finish_body,
      grid=(grid,),
      in_specs=[
          pl.BlockSpec((NC, blk, D), lambda i: (0, i, 0)),
          pl.BlockSpec((blk, NW), lambda i: (i, 0)),
          pl.BlockSpec((blk, D), lambda i: (i, 0)),
          pl.BlockSpec((D, D), lambda i: (0, 0)),
          pl.BlockSpec((1, D), lambda i: (0, 0)),
          pl.BlockSpec((D, D), lambda i: (0, 0)),
      ],
      out_specs=pl.BlockSpec((blk, D), lambda i: (i, 0)),
      out_shape=jax.ShapeDtypeStruct((N, D), jnp.float32),
  )(agg2, deg2.T, x, W_l, b_l2, W_r)


@jax.jit
def kernel(x, edge_index, W_l, b_l, W_r):
  epw_real = E // NW
  src = edge_index[0].reshape(NW, epw_real)
  dst = edge_index[1].reshape(NW, epw_real)
  src = jnp.pad(src, ((0, 0), (0, EPW - epw_real)))
  dst = jnp.pad(dst, ((0, 0), (0, EPW - epw_real)), constant_values=NP - 1)
  ei4 = jnp.stack([src, dst], axis=1).reshape(NW, 2, NCHUNK, CB)
  ei4 = ei4.transpose(0, 2, 1, 3)
  zeros = jnp.zeros((NP, D), jnp.float32)
  agg2, deg2 = _sc_aggregate(x, ei4, zeros)
  return _tc_finish(agg2, deg2, x, W_l, b_l.reshape(1, D), W_r)


# speedup vs baseline: 1.7624x; 1.7624x over previous
import functools

import jax
import jax.numpy as jnp
from jax import lax
from jax.experimental import pallas as pl
from jax.experimental.pallas import tpu as pltpu
from jax.experimental.pallas import tpu_sc as plsc

N = 10000
E = 320000
D = 128
DA = 144

NC = 2
NS = 16
NW = NC * NS
EPW = E // NW
CB = 50
NCHUNK = EPW // CB
NP = 10240
RPT = NP // NS


def _sc_aggregate(xaug, src3, dst3, zeros):
  mesh = plsc.VectorSubcoreMesh(core_axis_name="c", subcore_axis_name="s")

  @functools.partial(
      pl.kernel,
      out_type=jax.ShapeDtypeStruct((NC, NP, DA), jnp.float32),
      mesh=mesh,
      compiler_params=pltpu.CompilerParams(use_tc_tiling_on_sc=False,
                                           needs_layout_passes=False),
      scratch_types=[
          pltpu.VMEM((NCHUNK, CB), jnp.int32),
          pltpu.VMEM((NCHUNK, CB), jnp.int32),
          pltpu.VMEM((CB, DA), jnp.float32),
          pltpu.VMEM((CB, DA), jnp.float32),
          pltpu.VMEM_SHARED((NP, DA), jnp.float32),
          pltpu.SemaphoreType.DMA,
          pltpu.SemaphoreType.DMA,
      ],
  )
  def agg_kernel(x_hbm, src_hbm, dst_hbm, z_hbm, out_hbm,
                 src_v, dst_v, gbufa, gbufb, acc_sh, sema, semb):
    cid = lax.axis_index("c")
    sid = lax.axis_index("s")
    wid = cid * NS + sid

    pltpu.sync_copy(src_hbm.at[wid], src_v)
    pltpu.sync_copy(dst_hbm.at[wid], dst_v)

    pltpu.sync_copy(z_hbm.at[pl.ds(sid * RPT, RPT)],
                    acc_sh.at[pl.ds(sid * RPT, RPT)])
    plsc.subcore_barrier()

    pltpu.async_copy(x_hbm.at[src_v.at[0]], gbufa, sema)

    @pl.loop(0, NCHUNK, step=2)
    def _(j):
      hb = pltpu.async_copy(x_hbm.at[src_v.at[j + 1]], gbufb, semb)
      pltpu.make_async_copy(x_hbm.at[pl.ds(0, CB)], gbufa, sema).wait()
      pltpu.sync_copy(gbufa, acc_sh.at[dst_v.at[j]], add=True)

      @pl.when(j + 2 < NCHUNK)
      def _():
        pltpu.async_copy(x_hbm.at[src_v.at[j + 2]], gbufa, sema)

      hb.wait()
      pltpu.sync_copy(gbufb, acc_sh.at[dst_v.at[j + 1]], add=True)

    plsc.subcore_barrier()

    pltpu.sync_copy(acc_sh.at[pl.ds(sid * RPT, RPT)],
                    out_hbm.at[cid, pl.ds(sid * RPT, RPT)])

  return agg_kernel(xaug, src3, dst3, zeros)


def _tc_finish_body(agg_ref, x_ref, wl_ref, bl_ref, wr_ref, out_ref):
  a = agg_ref[0] + agg_ref[1]
  deg = a[:, D:D + 1]
  mean = a[:, :D] / jnp.maximum(deg, 1.0)
  out = (
      lax.dot_general(mean, wl_ref[...], (((1,), (1,)), ((), ())),
                      preferred_element_type=jnp.float32)
      + lax.dot_general(x_ref[...], wr_ref[...], (((1,), (1,)), ((), ())),
                        preferred_element_type=jnp.float32)
      + bl_ref[...]
  )
  norm = jnp.sqrt(jnp.sum(out * out, axis=-1, keepdims=True))
  out_ref[...] = out / jnp.maximum(norm, 1e-12)


def _tc_finish(agg2, x, W_l, b_l2, W_r):
  blk = 2000
  grid = N // blk
  return pl.pallas_call(
      _tc_finish_body,
      grid=(grid,),
      in_specs=[
          pl.BlockSpec((NC, blk, DA), lambda i: (0, i, 0)),
          pl.BlockSpec((blk, D), lambda i: (i, 0)),
          pl.BlockSpec((D, D), lambda i: (0, 0)),
          pl.BlockSpec((1, D), lambda i: (0, 0)),
          pl.BlockSpec((D, D), lambda i: (0, 0)),
      ],
      out_specs=pl.BlockSpec((blk, D), lambda i: (i, 0)),
      out_shape=jax.ShapeDtypeStruct((N, D), jnp.float32),
  )(agg2, x, W_l, b_l2, W_r)


@jax.jit
def kernel(x, edge_index, W_l, b_l, W_r):
  xaug = jnp.concatenate(
      [x, jnp.ones((N, 1), jnp.float32), jnp.zeros((N, DA - D - 1), jnp.float32)],
      axis=1)
  src3 = edge_index[0].reshape(NW, NCHUNK, CB)
  dst3 = edge_index[1].reshape(NW, NCHUNK, CB)
  zeros = jnp.zeros((NP, DA), jnp.float32)
  agg2 = _sc_aggregate(xaug, src3, dst3, zeros)
  return _tc_finish(agg2, x, W_l, b_l.reshape(1, D), W_r)


# speedup vs baseline: 2.0041x; 1.1372x over previous
import functools

import jax
import jax.numpy as jnp
from jax import lax
from jax.experimental import pallas as pl
from jax.experimental.pallas import tpu as pltpu
from jax.experimental.pallas import tpu_sc as plsc

N = 10000
E = 320000
D = 128

NC = 2
NS = 16
NW = NC * NS
EPW = E // NW
CB = 40
NCHUNK = EPW // CB
NP = 10240
RPT = NP // NS


def _sc_aggregate(x, src3, dst3, zeros, zeros1, ones):
  mesh = plsc.VectorSubcoreMesh(core_axis_name="c", subcore_axis_name="s")

  @functools.partial(
      pl.kernel,
      out_type=(jax.ShapeDtypeStruct((NC, NP, D), jnp.float32),
                jax.ShapeDtypeStruct((NC, NP, 16), jnp.float32)),
      mesh=mesh,
      compiler_params=pltpu.CompilerParams(use_tc_tiling_on_sc=False,
                                           needs_layout_passes=False),
      scratch_types=[
          pltpu.VMEM((NCHUNK, CB), jnp.int32),
          pltpu.VMEM((NCHUNK, CB), jnp.int32),
          pltpu.VMEM((CB, D), jnp.float32),
          pltpu.VMEM((CB, D), jnp.float32),
          pltpu.VMEM((CB, 16), jnp.float32),
          pltpu.VMEM_SHARED((NP, D), jnp.float32),
          pltpu.VMEM_SHARED((NP, 16), jnp.float32),
          pltpu.SemaphoreType.DMA,
          pltpu.SemaphoreType.DMA,
      ],
  )
  def agg_kernel(x_hbm, src_hbm, dst_hbm, z_hbm, z1_hbm, o_hbm, acc_hbm, deg_hbm,
                 src_v, dst_v, gbufa, gbufb, ones_v, acc_sh, deg_sh,
                 sema, semb):
    cid = lax.axis_index("c")
    sid = lax.axis_index("s")
    wid = cid * NS + sid

    pltpu.sync_copy(src_hbm.at[wid], src_v)
    pltpu.sync_copy(dst_hbm.at[wid], dst_v)

    pltpu.sync_copy(z_hbm.at[pl.ds(sid * RPT, RPT)],
                    acc_sh.at[pl.ds(sid * RPT, RPT)])
    pltpu.sync_copy(z1_hbm.at[pl.ds(sid * RPT, RPT)],
                    deg_sh.at[pl.ds(sid * RPT, RPT)])
    pltpu.sync_copy(o_hbm, ones_v)
    plsc.subcore_barrier()

    def scatter(gbuf, j):
      pltpu.sync_copy(gbuf, acc_sh.at[dst_v.at[j]], add=True)
      pltpu.sync_copy(ones_v, deg_sh.at[dst_v.at[j]], add=True)

    pltpu.async_copy(x_hbm.at[src_v.at[0]], gbufa, sema)

    @pl.loop(0, NCHUNK, step=2)
    def _(j):
      hb = pltpu.async_copy(x_hbm.at[src_v.at[j + 1]], gbufb, semb)
      pltpu.make_async_copy(x_hbm.at[pl.ds(0, CB)], gbufa, sema).wait()
      scatter(gbufa, j)

      @pl.when(j + 2 < NCHUNK)
      def _():
        pltpu.async_copy(x_hbm.at[src_v.at[j + 2]], gbufa, sema)

      hb.wait()
      scatter(gbufb, j + 1)

    plsc.subcore_barrier()

    pltpu.sync_copy(acc_sh.at[pl.ds(sid * RPT, RPT)],
                    acc_hbm.at[cid, pl.ds(sid * RPT, RPT)])
    pltpu.sync_copy(deg_sh.at[pl.ds(sid * RPT, RPT)],
                    deg_hbm.at[cid, pl.ds(sid * RPT, RPT)])

  return agg_kernel(x, src3, dst3, zeros, zeros1, ones)


def _tc_finish_body(agg_ref, deg_ref, x_ref, wl_ref, bl_ref, wr_ref, out_ref):
  a = agg_ref[0] + agg_ref[1]
  deg = jnp.sum(deg_ref[0] + deg_ref[1], axis=-1, keepdims=True)
  mean = a / jnp.maximum(deg, 1.0)
  out = (
      lax.dot_general(mean, wl_ref[...], (((1,), (1,)), ((), ())),
                      preferred_element_type=jnp.float32)
      + lax.dot_general(x_ref[...], wr_ref[...], (((1,), (1,)), ((), ())),
                        preferred_element_type=jnp.float32)
      + bl_ref[...]
  )
  norm = jnp.sqrt(jnp.sum(out * out, axis=-1, keepdims=True))
  out_ref[...] = out / jnp.maximum(norm, 1e-12)


def _tc_finish(agg2, deg2, x, W_l, b_l2, W_r):
  blk = 2000
  grid = N // blk
  return pl.pallas_call(
      _tc_finish_body,
      grid=(grid,),
      in_specs=[
          pl.BlockSpec((NC, blk, D), lambda i: (0, i, 0)),
          pl.BlockSpec((NC, blk, 16), lambda i: (0, i, 0)),
          pl.BlockSpec((blk, D), lambda i: (i, 0)),
          pl.BlockSpec((D, D), lambda i: (0, 0)),
          pl.BlockSpec((1, D), lambda i: (0, 0)),
          pl.BlockSpec((D, D), lambda i: (0, 0)),
      ],
      out_specs=pl.BlockSpec((blk, D), lambda i: (i, 0)),
      out_shape=jax.ShapeDtypeStruct((N, D), jnp.float32),
  )(agg2, deg2, x, W_l, b_l2, W_r)


@jax.jit
def kernel(x, edge_index, W_l, b_l, W_r):
  src3 = edge_index[0].reshape(NW, NCHUNK, CB)
  dst3 = edge_index[1].reshape(NW, NCHUNK, CB)
  zeros = jnp.zeros((NP, D), jnp.float32)
  zeros1 = jnp.zeros((NP, 16), jnp.float32)
  ones = jnp.zeros((CB, 16), jnp.float32).at[:, 0].set(1.0)
  agg2, deg2 = _sc_aggregate(x, src3, dst3, zeros, zeros1, ones)
  return _tc_finish(agg2, deg2, x, W_l, b_l.reshape(1, D), W_r)


# speedup vs baseline: 2.0241x; 1.0100x over previous
import functools

import jax
import jax.numpy as jnp
from jax import lax
from jax.experimental import pallas as pl
from jax.experimental.pallas import tpu as pltpu
from jax.experimental.pallas import tpu_sc as plsc

N = 10000
E = 320000
D = 128

NC = 2
NS = 16
NW = NC * NS
EPW = E // NW
CB = 50
NCHUNK = EPW // CB
NP = 10240
RPT = NP // NS


def _sc_aggregate(x, src3, dst3, zeros, zeros1, ones):
  mesh = plsc.VectorSubcoreMesh(core_axis_name="c", subcore_axis_name="s")

  @functools.partial(
      pl.kernel,
      out_type=(jax.ShapeDtypeStruct((NC, NP, D), jnp.float32),
                jax.ShapeDtypeStruct((NC, NP, 16), jnp.float32)),
      mesh=mesh,
      compiler_params=pltpu.CompilerParams(use_tc_tiling_on_sc=False,
                                           needs_layout_passes=False),
      scratch_types=[
          pltpu.VMEM((NCHUNK, CB), jnp.int32),
          pltpu.VMEM((NCHUNK, CB), jnp.int32),
          pltpu.VMEM((CB, D), jnp.float32),
          pltpu.VMEM((CB, D), jnp.float32),
          pltpu.VMEM((CB, 16), jnp.float32),
          pltpu.VMEM_SHARED((NP, D), jnp.float32),
          pltpu.VMEM_SHARED((NP, 16), jnp.float32),
          pltpu.SemaphoreType.DMA,
          pltpu.SemaphoreType.DMA,
          pltpu.SemaphoreType.DMA,
      ],
  )
  def agg_kernel(x_hbm, src_hbm, dst_hbm, z_hbm, z1_hbm, o_hbm, acc_hbm, deg_hbm,
                 src_v, dst_v, gbufa, gbufb, ones_v, acc_sh, deg_sh,
                 sema, semb, semd):
    cid = lax.axis_index("c")
    sid = lax.axis_index("s")
    wid = cid * NS + sid

    pltpu.sync_copy(src_hbm.at[wid], src_v)
    pltpu.sync_copy(dst_hbm.at[wid], dst_v)

    pltpu.sync_copy(z_hbm.at[pl.ds(sid * RPT, RPT)],
                    acc_sh.at[pl.ds(sid * RPT, RPT)])
    pltpu.sync_copy(z1_hbm.at[pl.ds(sid * RPT, RPT)],
                    deg_sh.at[pl.ds(sid * RPT, RPT)])
    pltpu.sync_copy(o_hbm, ones_v)
    plsc.subcore_barrier()

    def scatter(gbuf, j):
      pltpu.sync_copy(gbuf, acc_sh.at[dst_v.at[j]], add=True)
      pltpu.async_copy(ones_v, deg_sh.at[dst_v.at[j]], semd, add=True)

    pltpu.async_copy(x_hbm.at[src_v.at[0]], gbufa, sema)

    @pl.loop(0, NCHUNK, step=2)
    def _(j):
      hb = pltpu.async_copy(x_hbm.at[src_v.at[j + 1]], gbufb, semb)
      pltpu.make_async_copy(x_hbm.at[pl.ds(0, CB)], gbufa, sema).wait()
      scatter(gbufa, j)

      @pl.when(j + 2 < NCHUNK)
      def _():
        pltpu.async_copy(x_hbm.at[src_v.at[j + 2]], gbufa, sema)

      hb.wait()
      scatter(gbufb, j + 1)

    @pl.loop(0, NCHUNK)
    def _(j):
      pltpu.make_async_copy(z1_hbm.at[pl.ds(0, CB)], ones_v, semd).wait()

    plsc.subcore_barrier()

    pltpu.sync_copy(acc_sh.at[pl.ds(sid * RPT, RPT)],
                    acc_hbm.at[cid, pl.ds(sid * RPT, RPT)])
    pltpu.sync_copy(deg_sh.at[pl.ds(sid * RPT, RPT)],
                    deg_hbm.at[cid, pl.ds(sid * RPT, RPT)])

  return agg_kernel(x, src3, dst3, zeros, zeros1, ones)


def _tc_finish_body(agg_ref, deg_ref, x_ref, wl_ref, bl_ref, wr_ref, out_ref):
  a = agg_ref[0] + agg_ref[1]
  deg = jnp.sum(deg_ref[0] + deg_ref[1], axis=-1, keepdims=True)
  mean = a / jnp.maximum(deg, 1.0)
  out = (
      lax.dot_general(mean, wl_ref[...], (((1,), (1,)), ((), ())),
                      preferred_element_type=jnp.float32)
      + lax.dot_general(x_ref[...], wr_ref[...], (((1,), (1,)), ((), ())),
                        preferred_element_type=jnp.float32)
      + bl_ref[...]
  )
  norm = jnp.sqrt(jnp.sum(out * out, axis=-1, keepdims=True))
  out_ref[...] = out / jnp.maximum(norm, 1e-12)


def _tc_finish(agg2, deg2, x, W_l, b_l2, W_r):
  blk = 2000
  grid = N // blk
  return pl.pallas_call(
      _tc_finish_body,
      grid=(grid,),
      in_specs=[
          pl.BlockSpec((NC, blk, D), lambda i: (0, i, 0)),
          pl.BlockSpec((NC, blk, 16), lambda i: (0, i, 0)),
          pl.BlockSpec((blk, D), lambda i: (i, 0)),
          pl.BlockSpec((D, D), lambda i: (0, 0)),
          pl.BlockSpec((1, D), lambda i: (0, 0)),
          pl.BlockSpec((D, D), lambda i: (0, 0)),
      ],
      out_specs=pl.BlockSpec((blk, D), lambda i: (i, 0)),
      out_shape=jax.ShapeDtypeStruct((N, D), jnp.float32),
  )(agg2, deg2, x, W_l, b_l2, W_r)


@jax.jit
def kernel(x, edge_index, W_l, b_l, W_r):
  src3 = edge_index[0].reshape(NW, NCHUNK, CB)
  dst3 = edge_index[1].reshape(NW, NCHUNK, CB)
  zeros = jnp.zeros((NP, D), jnp.float32)
  zeros1 = jnp.zeros((NP, 16), jnp.float32)
  ones = jnp.zeros((CB, 16), jnp.float32).at[:, 0].set(1.0)
  agg2, deg2 = _sc_aggregate(x, src3, dst3, zeros, zeros1, ones)
  return _tc_finish(agg2, deg2, x, W_l, b_l.reshape(1, D), W_r)
